# baseline (speedup 1.0000x reference)
import functools

import jax
import jax.numpy as jnp
from jax import lax
from jax.experimental import pallas as pl
from jax.experimental.pallas import tpu as pltpu
from jax.experimental.pallas import tpu_sc as plsc

N = 10000
E = 160000
D_IN = 20
D = 512
CH = 128
NCH = D // CH
OUT_NUM = 10
EPS = 1e-5

NTILES = 32
EPT = E // NTILES
K = 100
NB = EPT // K
NBH = NB // 2
RPT = 632
RPT_LAST = N - 15 * RPT
ZCH = 96
ZT_A = RPT - 6 * ZCH
ZT_B = RPT_LAST - 5 * ZCH
CNTW = 16


@functools.lru_cache(maxsize=None)
def _build_conv():
    mesh = plsc.VectorSubcoreMesh(core_axis_name="c", subcore_axis_name="s")
    scratch = [
        pltpu.VMEM((NB, K), jnp.int32),
        pltpu.VMEM((NB, K), jnp.int32),
        pltpu.VMEM((K, CH), jnp.float32),
        pltpu.VMEM((K, CH), jnp.float32),
        pltpu.VMEM_SHARED((N, CH), jnp.float32),
        pltpu.SemaphoreType.DMA,
        pltpu.SemaphoreType.DMA,
    ]

    def body(h0_hbm, h1_hbm, h2_hbm, h3_hbm, src_hbm, dst_hbm, z_hbm,
             out_hbm, srcb, dstb, rows0, rows1, acc, g0, g1):
        c = lax.axis_index("c")
        s = lax.axis_index("s")
        tid = c * 16 + s
        r0 = s * RPT
        nfull = jnp.where(s < 15, 6, 5)
        pltpu.sync_copy(src_hbm.at[tid], srcb)
        pltpu.sync_copy(dst_hbm.at[tid], dstb)

        for chunk, h_hbm in enumerate((h0_hbm, h1_hbm, h2_hbm, h3_hbm)):
            pltpu.sync_copy(z_hbm.at[pl.ds(0, ZCH)], rows0.at[pl.ds(0, ZCH)])

            def zstep(j, carry):
                pltpu.sync_copy(rows0.at[pl.ds(0, ZCH)],
                                acc.at[pl.ds(r0 + j * ZCH, ZCH)])
                return carry

            lax.fori_loop(0, nfull, zstep, 0)

            @pl.when(s < 15)
            def _():
                pltpu.sync_copy(rows0.at[pl.ds(0, ZT_A)],
                                acc.at[pl.ds(r0 + 6 * ZCH, ZT_A)])

            @pl.when(s == 15)
            def _():
                pltpu.sync_copy(rows0.at[pl.ds(0, ZT_B)],
                                acc.at[pl.ds(r0 + 5 * ZCH, ZT_B)])

            plsc.subcore_barrier()

            pltpu.async_copy(h_hbm.at[srcb.at[0]], rows0, g0)

            def step(t, carry):
                j0 = 2 * t
                j1 = j0 + 1
                pltpu.make_async_copy(h_hbm.at[srcb.at[j0]], rows0, g0).wait()
                pltpu.async_copy(h_hbm.at[srcb.at[j1]], rows1, g1)
                pltpu.sync_copy(rows0, acc.at[dstb.at[j0]], add=True)
                pltpu.make_async_copy(h_hbm.at[srcb.at[j1]], rows1, g1).wait()

                @pl.when(t < NBH - 1)
                def _():
                    pltpu.async_copy(h_hbm.at[srcb.at[j0 + 2]], rows0, g0)

                pltpu.sync_copy(rows1, acc.at[dstb.at[j1]], add=True)
                return carry

            lax.fori_loop(0, NBH, step, 0)

            plsc.subcore_barrier()
            ob = chunk * 2 * N + c * N + s * RPT

            def wstep(j, carry):
                pltpu.sync_copy(acc.at[pl.ds(r0 + j * ZCH, ZCH)],
                                rows0.at[pl.ds(0, ZCH)])
                pltpu.sync_copy(rows0.at[pl.ds(0, ZCH)],
                                out_hbm.at[pl.ds(ob + j * ZCH, ZCH)])
                return carry

            lax.fori_loop(0, nfull, wstep, 0)

            @pl.when(s < 15)
            def _():
                pltpu.sync_copy(acc.at[pl.ds(r0 + 6 * ZCH, ZT_A)],
                                rows0.at[pl.ds(0, ZT_A)])
                pltpu.sync_copy(rows0.at[pl.ds(0, ZT_A)],
                                out_hbm.at[pl.ds(ob + 6 * ZCH, ZT_A)])

            @pl.when(s == 15)
            def _():
                pltpu.sync_copy(acc.at[pl.ds(r0 + 5 * ZCH, ZT_B)],
                                rows0.at[pl.ds(0, ZT_B)])
                pltpu.sync_copy(rows0.at[pl.ds(0, ZT_B)],
                                out_hbm.at[pl.ds(ob + 5 * ZCH, ZT_B)])

    return pl.kernel(
        body,
        out_type=jax.ShapeDtypeStruct((4 * 2 * N, CH), jnp.float32),
        mesh=mesh,
        scratch_types=scratch,
    )


@functools.lru_cache(maxsize=None)
def _build_count():
    mesh = plsc.VectorSubcoreMesh(core_axis_name="c", subcore_axis_name="s")
    scratch = [
        pltpu.VMEM((NB, K), jnp.int32),
        pltpu.VMEM((K, CNTW), jnp.float32),
        pltpu.VMEM_SHARED((N, CNTW), jnp.float32),
        pltpu.VMEM((ZCH, CNTW), jnp.float32),
    ]

    def body(dst_hbm, zc_hbm, ones_hbm, cnt_hbm,
             dstb, onesb, accc, cstage):
        c = lax.axis_index("c")
        s = lax.axis_index("s")
        tid = c * 16 + s
        r0 = s * RPT
        nfull = jnp.where(s < 15, 6, 5)
        pltpu.sync_copy(zc_hbm.at[pl.ds(0, ZCH)], cstage)
        pltpu.sync_copy(ones_hbm, onesb)
        pltpu.sync_copy(dst_hbm.at[tid], dstb)

        def zstep(j, carry):
            pltpu.sync_copy(cstage, accc.at[pl.ds(r0 + j * ZCH, ZCH)])
            return carry

        lax.fori_loop(0, nfull, zstep, 0)

        @pl.when(s < 15)
        def _():
            pltpu.sync_copy(cstage.at[pl.ds(0, ZT_A)],
                            accc.at[pl.ds(r0 + 6 * ZCH, ZT_A)])

        @pl.when(s == 15)
        def _():
            pltpu.sync_copy(cstage.at[pl.ds(0, ZT_B)],
                            accc.at[pl.ds(r0 + 5 * ZCH, ZT_B)])

        plsc.subcore_barrier()

        def step(i, carry):
            pltpu.sync_copy(onesb, accc.at[dstb.at[i]], add=True)
            return carry

        lax.fori_loop(0, NB, step, 0)

        plsc.subcore_barrier()
        ob = c * N + s * RPT

        def wstep(j, carry):
            pltpu.sync_copy(accc.at[pl.ds(r0 + j * ZCH, ZCH)], cstage)
            pltpu.sync_copy(cstage, cnt_hbm.at[pl.ds(ob + j * ZCH, ZCH)])
            return carry

        lax.fori_loop(0, nfull, wstep, 0)

        @pl.when(s < 15)
        def _():
            pltpu.sync_copy(accc.at[pl.ds(r0 + 6 * ZCH, ZT_A)],
                            cstage.at[pl.ds(0, ZT_A)])
            pltpu.sync_copy(cstage.at[pl.ds(0, ZT_A)],
                            cnt_hbm.at[pl.ds(ob + 6 * ZCH, ZT_A)])

        @pl.when(s == 15)
        def _():
            pltpu.sync_copy(accc.at[pl.ds(r0 + 5 * ZCH, ZT_B)],
                            cstage.at[pl.ds(0, ZT_B)])
            pltpu.sync_copy(cstage.at[pl.ds(0, ZT_B)],
                            cnt_hbm.at[pl.ds(ob + 5 * ZCH, ZT_B)])

    return pl.kernel(
        body,
        out_type=jax.ShapeDtypeStruct((2 * N, CNTW), jnp.float32),
        mesh=mesh,
        scratch_types=scratch,
    )


def _table_gather(table, ids16):
    mesh = plsc.VectorSubcoreMesh(core_axis_name="c", subcore_axis_name="s")

    def body(table_hbm, ids_hbm, out_hbm, idxv, rowsv, sem):
        w = lax.axis_index("s") * 2 + lax.axis_index("c")

        @pl.when(w == 0)
        def _():
            pltpu.sync_copy(ids_hbm, idxv)
            pltpu.async_copy(table_hbm.at[idxv], rowsv, sem).wait()
            pltpu.sync_copy(rowsv, out_hbm)

    k = pl.kernel(
        body,
        out_type=jax.ShapeDtypeStruct((16, D), jnp.float32),
        mesh=mesh,
        scratch_types=[
            pltpu.VMEM((16,), jnp.int32),
            pltpu.VMEM((16, D), jnp.float32),
            pltpu.SemaphoreType.DMA,
        ],
    )
    return k(table, ids16)


def _prologue(x, W, b):
    def body(x_ref, w_ref, b_ref, o_ref):
        o_ref[0] = (
            jnp.dot(x_ref[...], w_ref[...], preferred_element_type=jnp.float32)
            + b_ref[...]
        )

    return pl.pallas_call(
        body,
        grid=(NCH,),
        in_specs=[
            pl.BlockSpec((N, D_IN), lambda c: (0, 0)),
            pl.BlockSpec((D_IN, CH), lambda c: (0, c)),
            pl.BlockSpec((1, CH), lambda c: (0, c)),
        ],
        out_specs=pl.BlockSpec((1, N, CH), lambda c: (c, 0, 0)),
        out_shape=jax.ShapeDtypeStruct((NCH, N, CH), jnp.float32),
    )(x, W, b.reshape(1, D))


def _epilogue(h_c, part_c, cnt, gamma_c, beta_c, want_colsum):
    def body(h_ref, p_ref, c_ref, g_ref, bt_ref, *outs):
        agg = p_ref[0] + p_ref[1]
        cntv = c_ref[0, :, 0:1] + c_ref[1, :, 0:1]
        hnew = h_ref[...] + agg / jnp.maximum(cntv, 1.0)
        mu = jnp.mean(hnew, axis=0, keepdims=True)
        var = jnp.mean((hnew - mu) ** 2, axis=0, keepdims=True)
        y = (hnew - mu) * lax.rsqrt(var + EPS) * g_ref[...] + bt_ref[...]
        y = jnp.maximum(y, 0.0)
        outs[0][...] = y
        if want_colsum:
            outs[1][...] = jnp.sum(y, axis=0, keepdims=True)

    out_shape = [jax.ShapeDtypeStruct((N, CH), jnp.float32)]
    if want_colsum:
        out_shape.append(jax.ShapeDtypeStruct((1, CH), jnp.float32))
    res = pl.pallas_call(body, out_shape=out_shape)(
        h_c, part_c, cnt, gamma_c, beta_c)
    return res if want_colsum else res[0]


def _head(qemb, rows16, W_fc, b_fc, g1, bt1, W2, b2, g2, bt2,
          W3, b3, g3, bt3, W4, b4):
    def bnrelu(t, g, bt):
        mu = jnp.mean(t, axis=0, keepdims=True)
        var = jnp.mean((t - mu) ** 2, axis=0, keepdims=True)
        return jnp.maximum((t - mu) * lax.rsqrt(var + EPS) * g + bt, 0.0)

    def body(q_ref, r_ref, wfc_ref, bfc_ref, g1_ref, bt1_ref,
             w2_ref, b2_ref, g2_ref, bt2_ref,
             w3_ref, b3_ref, g3_ref, bt3_ref,
             w4_ref, b4_ref, o_ref):
        q = jnp.broadcast_to(q_ref[...], (OUT_NUM, D))
        pg = jnp.broadcast_to(r_ref[10:11, :], (OUT_NUM, D))
        ne = r_ref[0:10, :]
        b = jnp.concatenate([q, pg, ne], axis=1)
        h1 = bnrelu(
            jnp.dot(b, wfc_ref[...], preferred_element_type=jnp.float32)
            + bfc_ref[...], g1_ref[...], bt1_ref[...])
        h2 = bnrelu(
            jnp.dot(h1, w2_ref[...], preferred_element_type=jnp.float32)
            + b2_ref[...], g2_ref[...], bt2_ref[...])
        h3 = bnrelu(
            jnp.dot(h2, w3_ref[...], preferred_element_type=jnp.float32)
            + b3_ref[...], g3_ref[...], bt3_ref[...])
        logits = jnp.dot(h3, w4_ref[...], preferred_element_type=jnp.float32) \
            + b4_ref[...]
        o_ref[...] = jax.nn.sigmoid(logits)

    return pl.pallas_call(
        body,
        out_shape=jax.ShapeDtypeStruct((OUT_NUM, 1), jnp.float32),
    )(qemb, rows16, W_fc, b_fc.reshape(1, -1), g1.reshape(1, -1),
      bt1.reshape(1, -1), W2, b2.reshape(1, -1), g2.reshape(1, -1),
      bt2.reshape(1, -1), W3, b3.reshape(1, -1), g3.reshape(1, -1),
      bt3.reshape(1, -1), W4, b4.reshape(1, -1))


def _conv_layer(h_chunks, src3, dst3, zeros_big):
    p = _build_conv()(h_chunks[0], h_chunks[1], h_chunks[2], h_chunks[3],
                      src3, dst3, zeros_big)
    if isinstance(p, (tuple, list)):
        p = p[0]
    p = p.reshape(NCH, 2, N, CH)
    return [p[c] for c in range(NCH)]


def kernel(x, edge_index, neigh_ids, pg_ids, table, W_init, b_init,
           gamma1, beta1, gamma2, beta2,
           W_fc, b_fc, gamma_b1, beta_b1,
           W_fc2, b_fc2, gamma_b2, beta_b2,
           W_fc3, b_fc3, gamma_b3, beta_b3,
           W_fc4, b_fc4):
    src = edge_index[0]
    dst = edge_index[1]
    src3 = src.reshape(NTILES, NB, K)
    dst3 = dst.reshape(NTILES, NB, K)
    zeros_big = jnp.zeros((N, CH), jnp.float32)
    zeros_cnt = jnp.zeros((N, CNTW), jnp.float32)
    ones_arr = jnp.ones((K, CNTW), jnp.float32)

    h0 = _prologue(x, W_init, b_init)
    h0c = [h0[c] for c in range(NCH)]

    cnt_raw = _build_count()(dst3, zeros_cnt, ones_arr)
    if isinstance(cnt_raw, (tuple, list)):
        cnt_raw = cnt_raw[0]
    cnt = cnt_raw.reshape(2, N, CNTW)

    parts1 = _conv_layer(h0c, src3, dst3, zeros_big)
    h1c = []
    for c in range(NCH):
        g = gamma1[c * CH:(c + 1) * CH].reshape(1, CH)
        bt = beta1[c * CH:(c + 1) * CH].reshape(1, CH)
        h1c.append(_epilogue(h0c[c], parts1[c], cnt, g, bt, False))

    parts2 = _conv_layer(h1c, src3, dst3, zeros_big)
    h2c = []
    qs = []
    for c in range(NCH):
        g = gamma2[c * CH:(c + 1) * CH].reshape(1, CH)
        bt = beta2[c * CH:(c + 1) * CH].reshape(1, CH)
        y, s_ = _epilogue(h1c[c], parts2[c], cnt, g, bt, True)
        h2c.append(y)
        qs.append(s_)
    qemb = jnp.concatenate(qs, axis=1) / float(N)

    ids16 = jnp.concatenate(
        [neigh_ids, pg_ids, jnp.zeros((5,), jnp.int32)])
    rows16 = _table_gather(table, ids16)

    pred = _head(qemb, rows16, W_fc, b_fc, gamma_b1, beta_b1,
                 W_fc2, b_fc2, gamma_b2, beta_b2,
                 W_fc3, b_fc3, gamma_b3, beta_b3,
                 W_fc4, b_fc4)
    return pred.reshape(1, OUT_NUM)


# speedup vs baseline: 1.1031x; 1.1031x over previous
import functools

import jax
import jax.numpy as jnp
from jax import lax
from jax.experimental import pallas as pl
from jax.experimental.pallas import tpu as pltpu
from jax.experimental.pallas import tpu_sc as plsc

N = 10000
E = 160000
D_IN = 20
D = 512
CH = 128
NCH = D // CH
OUT_NUM = 10
EPS = 1e-5

NTILES = 32
EPT = E // NTILES
K = 100
NB = EPT // K
NBH = NB // 2
RPT = 632
RPT_LAST = N - 15 * RPT
ZCH = 96
ZT_A = RPT - 6 * ZCH
ZT_B = RPT_LAST - 5 * ZCH
CNTW = 16


@functools.lru_cache(maxsize=None)
def _build_conv():
    mesh = plsc.VectorSubcoreMesh(core_axis_name="c", subcore_axis_name="s")
    scratch = [
        pltpu.VMEM((NB, K), jnp.int32),
        pltpu.VMEM((NB, K), jnp.int32),
        pltpu.VMEM((K, CH), jnp.float32),
        pltpu.VMEM((K, CH), jnp.float32),
        pltpu.VMEM_SHARED((N, CH), jnp.float32),
        pltpu.SemaphoreType.DMA,
        pltpu.SemaphoreType.DMA,
    ]

    def body(h_hbm, src_hbm, dst_hbm, z_hbm,
             out_hbm, srcb, dstb, rows0, rows1, acc, g0, g1):
        c = lax.axis_index("c")
        s = lax.axis_index("s")
        tid = c * 16 + s
        r0 = s * RPT
        nfull = jnp.where(s < 15, 6, 5)
        pltpu.sync_copy(src_hbm.at[tid], srcb)
        pltpu.sync_copy(dst_hbm.at[tid], dstb)

        for chunk in range(1):
            pltpu.sync_copy(z_hbm.at[pl.ds(0, ZCH)], rows0.at[pl.ds(0, ZCH)])

            def zstep(j, carry):
                pltpu.sync_copy(rows0.at[pl.ds(0, ZCH)],
                                acc.at[pl.ds(r0 + j * ZCH, ZCH)])
                return carry

            lax.fori_loop(0, nfull, zstep, 0)

            @pl.when(s < 15)
            def _():
                pltpu.sync_copy(rows0.at[pl.ds(0, ZT_A)],
                                acc.at[pl.ds(r0 + 6 * ZCH, ZT_A)])

            @pl.when(s == 15)
            def _():
                pltpu.sync_copy(rows0.at[pl.ds(0, ZT_B)],
                                acc.at[pl.ds(r0 + 5 * ZCH, ZT_B)])

            plsc.subcore_barrier()

            pltpu.async_copy(h_hbm.at[srcb.at[0]], rows0, g0)

            def step(t, carry):
                j0 = 2 * t
                j1 = j0 + 1
                pltpu.make_async_copy(h_hbm.at[srcb.at[j0]], rows0, g0).wait()
                pltpu.async_copy(h_hbm.at[srcb.at[j1]], rows1, g1)
                pltpu.sync_copy(rows0, acc.at[dstb.at[j0]], add=True)
                pltpu.make_async_copy(h_hbm.at[srcb.at[j1]], rows1, g1).wait()

                @pl.when(t < NBH - 1)
                def _():
                    pltpu.async_copy(h_hbm.at[srcb.at[j0 + 2]], rows0, g0)

                pltpu.sync_copy(rows1, acc.at[dstb.at[j1]], add=True)
                return carry

            lax.fori_loop(0, NBH, step, 0)

            plsc.subcore_barrier()
            ob = chunk * 2 * N + c * N + s * RPT

            def wstep(j, carry):
                pltpu.sync_copy(acc.at[pl.ds(r0 + j * ZCH, ZCH)],
                                rows0.at[pl.ds(0, ZCH)])
                pltpu.sync_copy(rows0.at[pl.ds(0, ZCH)],
                                out_hbm.at[pl.ds(ob + j * ZCH, ZCH)])
                return carry

            lax.fori_loop(0, nfull, wstep, 0)

            @pl.when(s < 15)
            def _():
                pltpu.sync_copy(acc.at[pl.ds(r0 + 6 * ZCH, ZT_A)],
                                rows0.at[pl.ds(0, ZT_A)])
                pltpu.sync_copy(rows0.at[pl.ds(0, ZT_A)],
                                out_hbm.at[pl.ds(ob + 6 * ZCH, ZT_A)])

            @pl.when(s == 15)
            def _():
                pltpu.sync_copy(acc.at[pl.ds(r0 + 5 * ZCH, ZT_B)],
                                rows0.at[pl.ds(0, ZT_B)])
                pltpu.sync_copy(rows0.at[pl.ds(0, ZT_B)],
                                out_hbm.at[pl.ds(ob + 5 * ZCH, ZT_B)])

    return pl.kernel(
        body,
        out_type=jax.ShapeDtypeStruct((2 * N, CH), jnp.float32),
        mesh=mesh,
        scratch_types=scratch,
    )


@functools.lru_cache(maxsize=None)
def _build_count():
    mesh = plsc.VectorSubcoreMesh(core_axis_name="c", subcore_axis_name="s")
    scratch = [
        pltpu.VMEM((NB, K), jnp.int32),
        pltpu.VMEM((K, CNTW), jnp.float32),
        pltpu.VMEM_SHARED((N, CNTW), jnp.float32),
        pltpu.VMEM((ZCH, CNTW), jnp.float32),
    ]

    def body(dst_hbm, zc_hbm, ones_hbm, cnt_hbm,
             dstb, onesb, accc, cstage):
        c = lax.axis_index("c")
        s = lax.axis_index("s")
        tid = c * 16 + s
        r0 = s * RPT
        nfull = jnp.where(s < 15, 6, 5)
        pltpu.sync_copy(zc_hbm.at[pl.ds(0, ZCH)], cstage)
        pltpu.sync_copy(ones_hbm, onesb)
        pltpu.sync_copy(dst_hbm.at[tid], dstb)

        def zstep(j, carry):
            pltpu.sync_copy(cstage, accc.at[pl.ds(r0 + j * ZCH, ZCH)])
            return carry

        lax.fori_loop(0, nfull, zstep, 0)

        @pl.when(s < 15)
        def _():
            pltpu.sync_copy(cstage.at[pl.ds(0, ZT_A)],
                            accc.at[pl.ds(r0 + 6 * ZCH, ZT_A)])

        @pl.when(s == 15)
        def _():
            pltpu.sync_copy(cstage.at[pl.ds(0, ZT_B)],
                            accc.at[pl.ds(r0 + 5 * ZCH, ZT_B)])

        plsc.subcore_barrier()

        def step(i, carry):
            pltpu.sync_copy(onesb, accc.at[dstb.at[i]], add=True)
            return carry

        lax.fori_loop(0, NB, step, 0)

        plsc.subcore_barrier()
        ob = c * N + s * RPT

        def wstep(j, carry):
            pltpu.sync_copy(accc.at[pl.ds(r0 + j * ZCH, ZCH)], cstage)
            pltpu.sync_copy(cstage, cnt_hbm.at[pl.ds(ob + j * ZCH, ZCH)])
            return carry

        lax.fori_loop(0, nfull, wstep, 0)

        @pl.when(s < 15)
        def _():
            pltpu.sync_copy(accc.at[pl.ds(r0 + 6 * ZCH, ZT_A)],
                            cstage.at[pl.ds(0, ZT_A)])
            pltpu.sync_copy(cstage.at[pl.ds(0, ZT_A)],
                            cnt_hbm.at[pl.ds(ob + 6 * ZCH, ZT_A)])

        @pl.when(s == 15)
        def _():
            pltpu.sync_copy(accc.at[pl.ds(r0 + 5 * ZCH, ZT_B)],
                            cstage.at[pl.ds(0, ZT_B)])
            pltpu.sync_copy(cstage.at[pl.ds(0, ZT_B)],
                            cnt_hbm.at[pl.ds(ob + 5 * ZCH, ZT_B)])

    return pl.kernel(
        body,
        out_type=jax.ShapeDtypeStruct((2 * N, CNTW), jnp.float32),
        mesh=mesh,
        scratch_types=scratch,
    )


def _table_gather(table, ids16):
    mesh = plsc.VectorSubcoreMesh(core_axis_name="c", subcore_axis_name="s")

    def body(table_hbm, ids_hbm, out_hbm, idxv, rowsv, sem):
        w = lax.axis_index("s") * 2 + lax.axis_index("c")

        @pl.when(w == 0)
        def _():
            pltpu.sync_copy(ids_hbm, idxv)
            pltpu.async_copy(table_hbm.at[idxv], rowsv, sem).wait()
            pltpu.sync_copy(rowsv, out_hbm)

    k = pl.kernel(
        body,
        out_type=jax.ShapeDtypeStruct((16, D), jnp.float32),
        mesh=mesh,
        scratch_types=[
            pltpu.VMEM((16,), jnp.int32),
            pltpu.VMEM((16, D), jnp.float32),
            pltpu.SemaphoreType.DMA,
        ],
    )
    return k(table, ids16)


def _prologue(x, W, b):
    def body(x_ref, w_ref, b_ref, o_ref):
        o_ref[0] = (
            jnp.dot(x_ref[...], w_ref[...], preferred_element_type=jnp.float32)
            + b_ref[...]
        )

    return pl.pallas_call(
        body,
        grid=(NCH,),
        in_specs=[
            pl.BlockSpec((N, D_IN), lambda c: (0, 0)),
            pl.BlockSpec((D_IN, CH), lambda c: (0, c)),
            pl.BlockSpec((1, CH), lambda c: (0, c)),
        ],
        out_specs=pl.BlockSpec((1, N, CH), lambda c: (c, 0, 0)),
        out_shape=jax.ShapeDtypeStruct((NCH, N, CH), jnp.float32),
    )(x, W, b.reshape(1, D))


def _epilogue(h_c, part_c, cnt, gamma_c, beta_c, want_colsum):
    def body(h_ref, p_ref, c_ref, g_ref, bt_ref, *outs):
        agg = p_ref[0] + p_ref[1]
        cntv = c_ref[0, :, 0:1] + c_ref[1, :, 0:1]
        hnew = h_ref[...] + agg / jnp.maximum(cntv, 1.0)
        mu = jnp.mean(hnew, axis=0, keepdims=True)
        var = jnp.mean((hnew - mu) ** 2, axis=0, keepdims=True)
        y = (hnew - mu) * lax.rsqrt(var + EPS) * g_ref[...] + bt_ref[...]
        y = jnp.maximum(y, 0.0)
        outs[0][...] = y
        if want_colsum:
            outs[1][...] = jnp.sum(y, axis=0, keepdims=True)

    out_shape = [jax.ShapeDtypeStruct((N, CH), jnp.float32)]
    if want_colsum:
        out_shape.append(jax.ShapeDtypeStruct((1, CH), jnp.float32))
    res = pl.pallas_call(body, out_shape=out_shape)(
        h_c, part_c, cnt, gamma_c, beta_c)
    return res if want_colsum else res[0]


def _head(qemb, rows16, W_fc, b_fc, g1, bt1, W2, b2, g2, bt2,
          W3, b3, g3, bt3, W4, b4):
    def bnrelu(t, g, bt):
        mu = jnp.mean(t, axis=0, keepdims=True)
        var = jnp.mean((t - mu) ** 2, axis=0, keepdims=True)
        return jnp.maximum((t - mu) * lax.rsqrt(var + EPS) * g + bt, 0.0)

    def body(q_ref, r_ref, wfc_ref, bfc_ref, g1_ref, bt1_ref,
             w2_ref, b2_ref, g2_ref, bt2_ref,
             w3_ref, b3_ref, g3_ref, bt3_ref,
             w4_ref, b4_ref, o_ref):
        q = jnp.broadcast_to(q_ref[...], (OUT_NUM, D))
        pg = jnp.broadcast_to(r_ref[10:11, :], (OUT_NUM, D))
        ne = r_ref[0:10, :]
        b = jnp.concatenate([q, pg, ne], axis=1)
        h1 = bnrelu(
            jnp.dot(b, wfc_ref[...], preferred_element_type=jnp.float32)
            + bfc_ref[...], g1_ref[...], bt1_ref[...])
        h2 = bnrelu(
            jnp.dot(h1, w2_ref[...], preferred_element_type=jnp.float32)
            + b2_ref[...], g2_ref[...], bt2_ref[...])
        h3 = bnrelu(
            jnp.dot(h2, w3_ref[...], preferred_element_type=jnp.float32)
            + b3_ref[...], g3_ref[...], bt3_ref[...])
        logits = jnp.dot(h3, w4_ref[...], preferred_element_type=jnp.float32) \
            + b4_ref[...]
        o_ref[...] = jax.nn.sigmoid(logits)

    return pl.pallas_call(
        body,
        out_shape=jax.ShapeDtypeStruct((OUT_NUM, 1), jnp.float32),
    )(qemb, rows16, W_fc, b_fc.reshape(1, -1), g1.reshape(1, -1),
      bt1.reshape(1, -1), W2, b2.reshape(1, -1), g2.reshape(1, -1),
      bt2.reshape(1, -1), W3, b3.reshape(1, -1), g3.reshape(1, -1),
      bt3.reshape(1, -1), W4, b4.reshape(1, -1))


def _conv_layer(h_chunks, src3, dst3, zeros_big):
    parts = []
    for c in range(NCH):
        p = _build_conv()(h_chunks[c], src3, dst3, zeros_big)
        if isinstance(p, (tuple, list)):
            p = p[0]
        parts.append(p.reshape(2, N, CH))
    return parts


def kernel(x, edge_index, neigh_ids, pg_ids, table, W_init, b_init,
           gamma1, beta1, gamma2, beta2,
           W_fc, b_fc, gamma_b1, beta_b1,
           W_fc2, b_fc2, gamma_b2, beta_b2,
           W_fc3, b_fc3, gamma_b3, beta_b3,
           W_fc4, b_fc4):
    src = edge_index[0]
    dst = edge_index[1]
    src3 = src.reshape(NTILES, NB, K)
    dst3 = dst.reshape(NTILES, NB, K)
    zeros_big = jnp.zeros((N, CH), jnp.float32)
    zeros_cnt = jnp.zeros((N, CNTW), jnp.float32)
    ones_arr = jnp.ones((K, CNTW), jnp.float32)

    h0 = _prologue(x, W_init, b_init)
    h0c = [h0[c] for c in range(NCH)]

    cnt_raw = _build_count()(dst3, zeros_cnt, ones_arr)
    if isinstance(cnt_raw, (tuple, list)):
        cnt_raw = cnt_raw[0]
    cnt = cnt_raw.reshape(2, N, CNTW)

    parts1 = _conv_layer(h0c, src3, dst3, zeros_big)
    h1c = []
    for c in range(NCH):
        g = gamma1[c * CH:(c + 1) * CH].reshape(1, CH)
        bt = beta1[c * CH:(c + 1) * CH].reshape(1, CH)
        h1c.append(_epilogue(h0c[c], parts1[c], cnt, g, bt, False))

    parts2 = _conv_layer(h1c, src3, dst3, zeros_big)
    h2c = []
    qs = []
    for c in range(NCH):
        g = gamma2[c * CH:(c + 1) * CH].reshape(1, CH)
        bt = beta2[c * CH:(c + 1) * CH].reshape(1, CH)
        y, s_ = _epilogue(h1c[c], parts2[c], cnt, g, bt, True)
        h2c.append(y)
        qs.append(s_)
    qemb = jnp.concatenate(qs, axis=1) / float(N)

    ids16 = jnp.concatenate(
        [neigh_ids, pg_ids, jnp.zeros((5,), jnp.int32)])
    rows16 = _table_gather(table, ids16)

    pred = _head(qemb, rows16, W_fc, b_fc, gamma_b1, beta_b1,
                 W_fc2, b_fc2, gamma_b2, beta_b2,
                 W_fc3, b_fc3, gamma_b3, beta_b3,
                 W_fc4, b_fc4)
    return pred.reshape(1, OUT_NUM)


# speedup vs baseline: 1.1292x; 1.0237x over previous
import functools

import jax
import jax.numpy as jnp
from jax import lax
from jax.experimental import pallas as pl
from jax.experimental.pallas import tpu as pltpu
from jax.experimental.pallas import tpu_sc as plsc

N = 10000
E = 160000
D_IN = 20
D = 512
CH = 128
NCH = D // CH
OUT_NUM = 10
EPS = 1e-5

NTILES = 32
EPT = E // NTILES
K = 100
NB = EPT // K
NBH = NB // 2
RPT = 632
RPT_LAST = N - 15 * RPT
ZCH = 96
ZT_A = RPT - 6 * ZCH
ZT_B = RPT_LAST - 5 * ZCH
CNTW = 16


@functools.lru_cache(maxsize=None)
def _build_conv():
    mesh = plsc.VectorSubcoreMesh(core_axis_name="c", subcore_axis_name="s")
    scratch = [
        pltpu.VMEM((NB, K), jnp.int32),
        pltpu.VMEM((NB, K), jnp.int32),
        pltpu.VMEM((K, CH), jnp.float32),
        pltpu.VMEM((K, CH), jnp.float32),
        pltpu.VMEM_SHARED((N, CH), jnp.float32),
        pltpu.SemaphoreType.DMA,
        pltpu.SemaphoreType.DMA,
    ]

    def body(h_hbm, src_hbm, dst_hbm, z_hbm,
             out_hbm, srcb, dstb, rows0, rows1, acc, g0, g1):
        c = lax.axis_index("c")
        s = lax.axis_index("s")
        tid = c * 16 + s
        r0 = s * RPT
        nfull = jnp.where(s < 15, 6, 5)
        pltpu.sync_copy(src_hbm.at[tid], srcb)
        pltpu.sync_copy(dst_hbm.at[tid], dstb)

        for chunk in range(1):
            pltpu.sync_copy(z_hbm.at[pl.ds(0, ZCH)], rows0.at[pl.ds(0, ZCH)])

            def zstep(j, carry):
                pltpu.async_copy(rows0.at[pl.ds(0, ZCH)],
                                 acc.at[pl.ds(r0 + j * ZCH, ZCH)], g0)
                return carry

            lax.fori_loop(0, nfull, zstep, 0)

            @pl.when(s < 15)
            def _():
                pltpu.async_copy(rows0.at[pl.ds(0, ZT_A)],
                                 acc.at[pl.ds(r0 + 6 * ZCH, ZT_A)], g1)

            @pl.when(s == 15)
            def _():
                pltpu.async_copy(rows0.at[pl.ds(0, ZT_B)],
                                 acc.at[pl.ds(r0 + 5 * ZCH, ZT_B)], g1)

            def zdrain(j, carry):
                pltpu.make_async_copy(rows0.at[pl.ds(0, ZCH)],
                                      acc.at[pl.ds(r0, ZCH)], g0).wait()
                return carry

            lax.fori_loop(0, nfull, zdrain, 0)

            @pl.when(s < 15)
            def _():
                pltpu.make_async_copy(rows0.at[pl.ds(0, ZT_A)],
                                      acc.at[pl.ds(r0, ZT_A)], g1).wait()

            @pl.when(s == 15)
            def _():
                pltpu.make_async_copy(rows0.at[pl.ds(0, ZT_B)],
                                      acc.at[pl.ds(r0, ZT_B)], g1).wait()

            plsc.subcore_barrier()

            pltpu.async_copy(h_hbm.at[srcb.at[0]], rows0, g0)

            def step(t, carry):
                j0 = 2 * t
                j1 = j0 + 1
                pltpu.make_async_copy(h_hbm.at[srcb.at[j0]], rows0, g0).wait()
                pltpu.async_copy(h_hbm.at[srcb.at[j1]], rows1, g1)
                pltpu.sync_copy(rows0, acc.at[dstb.at[j0]], add=True)
                pltpu.make_async_copy(h_hbm.at[srcb.at[j1]], rows1, g1).wait()
                pltpu.async_copy(h_hbm.at[srcb.at[j0 + 2]], rows0, g0)
                pltpu.sync_copy(rows1, acc.at[dstb.at[j1]], add=True)
                return carry

            lax.fori_loop(0, NBH - 1, step, 0)

            jL = NB - 2
            pltpu.make_async_copy(h_hbm.at[srcb.at[jL]], rows0, g0).wait()
            pltpu.async_copy(h_hbm.at[srcb.at[jL + 1]], rows1, g1)
            pltpu.sync_copy(rows0, acc.at[dstb.at[jL]], add=True)
            pltpu.make_async_copy(h_hbm.at[srcb.at[jL + 1]], rows1, g1).wait()
            pltpu.sync_copy(rows1, acc.at[dstb.at[jL + 1]], add=True)

            plsc.subcore_barrier()
            ob = chunk * 2 * N + c * N + s * RPT
            bufs = (rows0, rows1)
            sems = (g0, g1)
            for j in range(6):
                b = bufs[j % 2]
                sm = sems[j % 2]

                @pl.when(j < nfull)
                def _(j=j, b=b, sm=sm):
                    if j >= 2:
                        pltpu.make_async_copy(
                            b.at[pl.ds(0, ZCH)],
                            out_hbm.at[pl.ds(ob + (j - 2) * ZCH, ZCH)],
                            sm).wait()
                    pltpu.sync_copy(acc.at[pl.ds(r0 + j * ZCH, ZCH)],
                                    b.at[pl.ds(0, ZCH)])
                    pltpu.async_copy(b.at[pl.ds(0, ZCH)],
                                     out_hbm.at[pl.ds(ob + j * ZCH, ZCH)], sm)

            def wdrain(b, sm):
                pltpu.make_async_copy(b.at[pl.ds(0, ZCH)],
                                      out_hbm.at[pl.ds(ob, ZCH)], sm).wait()

            wdrain(rows0, g0)
            wdrain(rows1, g1)

            @pl.when(s < 15)
            def _():
                pltpu.sync_copy(acc.at[pl.ds(r0 + 6 * ZCH, ZT_A)],
                                rows0.at[pl.ds(0, ZT_A)])
                pltpu.sync_copy(rows0.at[pl.ds(0, ZT_A)],
                                out_hbm.at[pl.ds(ob + 6 * ZCH, ZT_A)])

            @pl.when(s == 15)
            def _():
                pltpu.sync_copy(acc.at[pl.ds(r0 + 5 * ZCH, ZT_B)],
                                rows0.at[pl.ds(0, ZT_B)])
                pltpu.sync_copy(rows0.at[pl.ds(0, ZT_B)],
                                out_hbm.at[pl.ds(ob + 5 * ZCH, ZT_B)])

    return pl.kernel(
        body,
        out_type=jax.ShapeDtypeStruct((2 * N, CH), jnp.float32),
        mesh=mesh,
        scratch_types=scratch,
    )


@functools.lru_cache(maxsize=None)
def _build_count():
    mesh = plsc.VectorSubcoreMesh(core_axis_name="c", subcore_axis_name="s")
    scratch = [
        pltpu.VMEM((NB, K), jnp.int32),
        pltpu.VMEM((K, CNTW), jnp.float32),
        pltpu.VMEM_SHARED((N, CNTW), jnp.float32),
        pltpu.VMEM((ZCH, CNTW), jnp.float32),
        pltpu.SemaphoreType.DMA,
    ]

    def body(dst_hbm, zc_hbm, ones_hbm, cnt_hbm,
             dstb, onesb, accc, cstage, sem):
        c = lax.axis_index("c")
        s = lax.axis_index("s")
        tid = c * 16 + s
        r0 = s * RPT
        nfull = jnp.where(s < 15, 6, 5)
        pltpu.sync_copy(zc_hbm.at[pl.ds(0, ZCH)], cstage)
        pltpu.sync_copy(ones_hbm, onesb)
        pltpu.sync_copy(dst_hbm.at[tid], dstb)

        def zstep(j, carry):
            pltpu.sync_copy(cstage, accc.at[pl.ds(r0 + j * ZCH, ZCH)])
            return carry

        lax.fori_loop(0, nfull, zstep, 0)

        @pl.when(s < 15)
        def _():
            pltpu.sync_copy(cstage.at[pl.ds(0, ZT_A)],
                            accc.at[pl.ds(r0 + 6 * ZCH, ZT_A)])

        @pl.when(s == 15)
        def _():
            pltpu.sync_copy(cstage.at[pl.ds(0, ZT_B)],
                            accc.at[pl.ds(r0 + 5 * ZCH, ZT_B)])

        plsc.subcore_barrier()

        def rnd(r, carry):
            def issue(i, carry2):
                pltpu.async_copy(onesb, accc.at[dstb.at[r * 10 + i]], sem,
                                 add=True)
                return carry2

            lax.fori_loop(0, 10, issue, 0)

            def drain(i, carry2):
                pltpu.make_async_copy(onesb, accc.at[dstb.at[0]], sem).wait()
                return carry2

            lax.fori_loop(0, 10, drain, 0)
            return carry

        lax.fori_loop(0, NB // 10, rnd, 0)

        plsc.subcore_barrier()
        ob = c * N + s * RPT

        def wstep(j, carry):
            pltpu.sync_copy(accc.at[pl.ds(r0 + j * ZCH, ZCH)], cstage)
            pltpu.sync_copy(cstage, cnt_hbm.at[pl.ds(ob + j * ZCH, ZCH)])
            return carry

        lax.fori_loop(0, nfull, wstep, 0)

        @pl.when(s < 15)
        def _():
            pltpu.sync_copy(accc.at[pl.ds(r0 + 6 * ZCH, ZT_A)],
                            cstage.at[pl.ds(0, ZT_A)])
            pltpu.sync_copy(cstage.at[pl.ds(0, ZT_A)],
                            cnt_hbm.at[pl.ds(ob + 6 * ZCH, ZT_A)])

        @pl.when(s == 15)
        def _():
            pltpu.sync_copy(accc.at[pl.ds(r0 + 5 * ZCH, ZT_B)],
                            cstage.at[pl.ds(0, ZT_B)])
            pltpu.sync_copy(cstage.at[pl.ds(0, ZT_B)],
                            cnt_hbm.at[pl.ds(ob + 5 * ZCH, ZT_B)])

    return pl.kernel(
        body,
        out_type=jax.ShapeDtypeStruct((2 * N, CNTW), jnp.float32),
        mesh=mesh,
        scratch_types=scratch,
    )


def _table_gather(table, ids16):
    mesh = plsc.VectorSubcoreMesh(core_axis_name="c", subcore_axis_name="s")

    def body(table_hbm, ids_hbm, out_hbm, idxv, rowsv, sem):
        w = lax.axis_index("s") * 2 + lax.axis_index("c")

        @pl.when(w == 0)
        def _():
            pltpu.sync_copy(ids_hbm, idxv)
            pltpu.async_copy(table_hbm.at[idxv], rowsv, sem).wait()
            pltpu.sync_copy(rowsv, out_hbm)

    k = pl.kernel(
        body,
        out_type=jax.ShapeDtypeStruct((16, D), jnp.float32),
        mesh=mesh,
        scratch_types=[
            pltpu.VMEM((16,), jnp.int32),
            pltpu.VMEM((16, D), jnp.float32),
            pltpu.SemaphoreType.DMA,
        ],
    )
    return k(table, ids16)


def _prologue(x, W, b):
    def body(x_ref, w_ref, b_ref, o_ref):
        o_ref[0] = (
            jnp.dot(x_ref[...], w_ref[...], preferred_element_type=jnp.float32)
            + b_ref[...]
        )

    return pl.pallas_call(
        body,
        grid=(NCH,),
        in_specs=[
            pl.BlockSpec((N, D_IN), lambda c: (0, 0)),
            pl.BlockSpec((D_IN, CH), lambda c: (0, c)),
            pl.BlockSpec((1, CH), lambda c: (0, c)),
        ],
        out_specs=pl.BlockSpec((1, N, CH), lambda c: (c, 0, 0)),
        out_shape=jax.ShapeDtypeStruct((NCH, N, CH), jnp.float32),
    )(x, W, b.reshape(1, D))


def _epilogue(h_c, part_c, cnt, gamma_c, beta_c, want_colsum):
    def body(h_ref, p_ref, c_ref, g_ref, bt_ref, *outs):
        agg = p_ref[0] + p_ref[1]
        cntv = c_ref[0, :, 0:1] + c_ref[1, :, 0:1]
        hnew = h_ref[...] + agg / jnp.maximum(cntv, 1.0)
        mu = jnp.mean(hnew, axis=0, keepdims=True)
        var = jnp.mean((hnew - mu) ** 2, axis=0, keepdims=True)
        y = (hnew - mu) * lax.rsqrt(var + EPS) * g_ref[...] + bt_ref[...]
        y = jnp.maximum(y, 0.0)
        outs[0][...] = y
        if want_colsum:
            outs[1][...] = jnp.sum(y, axis=0, keepdims=True)

    out_shape = [jax.ShapeDtypeStruct((N, CH), jnp.float32)]
    if want_colsum:
        out_shape.append(jax.ShapeDtypeStruct((1, CH), jnp.float32))
    res = pl.pallas_call(body, out_shape=out_shape)(
        h_c, part_c, cnt, gamma_c, beta_c)
    return res if want_colsum else res[0]


def _head(qemb, rows16, W_fc, b_fc, g1, bt1, W2, b2, g2, bt2,
          W3, b3, g3, bt3, W4, b4):
    def bnrelu(t, g, bt):
        mu = jnp.mean(t, axis=0, keepdims=True)
        var = jnp.mean((t - mu) ** 2, axis=0, keepdims=True)
        return jnp.maximum((t - mu) * lax.rsqrt(var + EPS) * g + bt, 0.0)

    def body(q_ref, r_ref, wfc_ref, bfc_ref, g1_ref, bt1_ref,
             w2_ref, b2_ref, g2_ref, bt2_ref,
             w3_ref, b3_ref, g3_ref, bt3_ref,
             w4_ref, b4_ref, o_ref):
        q = jnp.broadcast_to(q_ref[...], (OUT_NUM, D))
        pg = jnp.broadcast_to(r_ref[10:11, :], (OUT_NUM, D))
        ne = r_ref[0:10, :]
        b = jnp.concatenate([q, pg, ne], axis=1)
        h1 = bnrelu(
            jnp.dot(b, wfc_ref[...], preferred_element_type=jnp.float32)
            + bfc_ref[...], g1_ref[...], bt1_ref[...])
        h2 = bnrelu(
            jnp.dot(h1, w2_ref[...], preferred_element_type=jnp.float32)
            + b2_ref[...], g2_ref[...], bt2_ref[...])
        h3 = bnrelu(
            jnp.dot(h2, w3_ref[...], preferred_element_type=jnp.float32)
            + b3_ref[...], g3_ref[...], bt3_ref[...])
        logits = jnp.dot(h3, w4_ref[...], preferred_element_type=jnp.float32) \
            + b4_ref[...]
        o_ref[...] = jax.nn.sigmoid(logits)

    return pl.pallas_call(
        body,
        out_shape=jax.ShapeDtypeStruct((OUT_NUM, 1), jnp.float32),
    )(qemb, rows16, W_fc, b_fc.reshape(1, -1), g1.reshape(1, -1),
      bt1.reshape(1, -1), W2, b2.reshape(1, -1), g2.reshape(1, -1),
      bt2.reshape(1, -1), W3, b3.reshape(1, -1), g3.reshape(1, -1),
      bt3.reshape(1, -1), W4, b4.reshape(1, -1))


def _conv_layer(h_chunks, src3, dst3, zeros_big):
    parts = []
    for c in range(NCH):
        p = _build_conv()(h_chunks[c], src3, dst3, zeros_big)
        if isinstance(p, (tuple, list)):
            p = p[0]
        parts.append(p.reshape(2, N, CH))
    return parts


def kernel(x, edge_index, neigh_ids, pg_ids, table, W_init, b_init,
           gamma1, beta1, gamma2, beta2,
           W_fc, b_fc, gamma_b1, beta_b1,
           W_fc2, b_fc2, gamma_b2, beta_b2,
           W_fc3, b_fc3, gamma_b3, beta_b3,
           W_fc4, b_fc4):
    src = edge_index[0]
    dst = edge_index[1]
    src3 = src.reshape(NTILES, NB, K)
    dst3 = dst.reshape(NTILES, NB, K)
    zeros_big = jnp.zeros((N, CH), jnp.float32)
    zeros_cnt = jnp.zeros((N, CNTW), jnp.float32)
    ones_arr = jnp.ones((K, CNTW), jnp.float32)

    h0 = _prologue(x, W_init, b_init)
    h0c = [h0[c] for c in range(NCH)]

    cnt_raw = _build_count()(dst3, zeros_cnt, ones_arr)
    if isinstance(cnt_raw, (tuple, list)):
        cnt_raw = cnt_raw[0]
    cnt = cnt_raw.reshape(2, N, CNTW)

    parts1 = _conv_layer(h0c, src3, dst3, zeros_big)
    h1c = []
    for c in range(NCH):
        g = gamma1[c * CH:(c + 1) * CH].reshape(1, CH)
        bt = beta1[c * CH:(c + 1) * CH].reshape(1, CH)
        h1c.append(_epilogue(h0c[c], parts1[c], cnt, g, bt, False))

    parts2 = _conv_layer(h1c, src3, dst3, zeros_big)
    h2c = []
    qs = []
    for c in range(NCH):
        g = gamma2[c * CH:(c + 1) * CH].reshape(1, CH)
        bt = beta2[c * CH:(c + 1) * CH].reshape(1, CH)
        y, s_ = _epilogue(h1c[c], parts2[c], cnt, g, bt, True)
        h2c.append(y)
        qs.append(s_)
    qemb = jnp.concatenate(qs, axis=1) / float(N)

    ids16 = jnp.concatenate(
        [neigh_ids, pg_ids, jnp.zeros((5,), jnp.int32)])
    rows16 = _table_gather(table, ids16)

    pred = _head(qemb, rows16, W_fc, b_fc, gamma_b1, beta_b1,
                 W_fc2, b_fc2, gamma_b2, beta_b2,
                 W_fc3, b_fc3, gamma_b3, beta_b3,
                 W_fc4, b_fc4)
    return pred.reshape(1, OUT_NUM)


# speedup vs baseline: 1.1310x; 1.0016x over previous
import functools

import jax
import jax.numpy as jnp
from jax import lax
from jax.experimental import pallas as pl
from jax.experimental.pallas import tpu as pltpu
from jax.experimental.pallas import tpu_sc as plsc

N = 10000
E = 160000
D_IN = 20
D = 512
CH = 128
NCH = D // CH
OUT_NUM = 10
EPS = 1e-5

NTILES = 32
EPT = E // NTILES
K = 100
NB = EPT // K
NBH = NB // 2
RPT = 632
RPT_LAST = N - 15 * RPT
ZCH = 96
ZT_A = RPT - 6 * ZCH
ZT_B = RPT_LAST - 5 * ZCH
CNTW = 16


@functools.lru_cache(maxsize=None)
def _build_conv():
    mesh = plsc.VectorSubcoreMesh(core_axis_name="c", subcore_axis_name="s")
    scratch = [
        pltpu.VMEM((NB, K), jnp.int32),
        pltpu.VMEM((NB, K), jnp.int32),
        pltpu.VMEM((K, CH), jnp.float32),
        pltpu.VMEM((K, CH), jnp.float32),
        pltpu.VMEM_SHARED((N, CH), jnp.float32),
        pltpu.SemaphoreType.DMA,
        pltpu.SemaphoreType.DMA,
    ]

    def body(h_hbm, src_hbm, dst_hbm, z_hbm,
             out_hbm, srcb, dstb, rows0, rows1, acc, g0, g1):
        c = lax.axis_index("c")
        s = lax.axis_index("s")
        tid = c * 16 + s
        r0 = s * RPT
        nfull = jnp.where(s < 15, 6, 5)
        pltpu.sync_copy(src_hbm.at[tid], srcb)
        pltpu.sync_copy(dst_hbm.at[tid], dstb)

        for chunk in range(1):
            pltpu.sync_copy(z_hbm.at[pl.ds(0, ZCH)], rows0.at[pl.ds(0, ZCH)])

            def zstep(j, carry):
                pltpu.async_copy(rows0.at[pl.ds(0, ZCH)],
                                 acc.at[pl.ds(r0 + j * ZCH, ZCH)], g0)
                return carry

            lax.fori_loop(0, nfull, zstep, 0)

            @pl.when(s < 15)
            def _():
                pltpu.async_copy(rows0.at[pl.ds(0, ZT_A)],
                                 acc.at[pl.ds(r0 + 6 * ZCH, ZT_A)], g1)

            @pl.when(s == 15)
            def _():
                pltpu.async_copy(rows0.at[pl.ds(0, ZT_B)],
                                 acc.at[pl.ds(r0 + 5 * ZCH, ZT_B)], g1)

            def zdrain(j, carry):
                pltpu.make_async_copy(rows0.at[pl.ds(0, ZCH)],
                                      acc.at[pl.ds(r0, ZCH)], g0).wait()
                return carry

            lax.fori_loop(0, nfull, zdrain, 0)

            @pl.when(s < 15)
            def _():
                pltpu.make_async_copy(rows0.at[pl.ds(0, ZT_A)],
                                      acc.at[pl.ds(r0, ZT_A)], g1).wait()

            @pl.when(s == 15)
            def _():
                pltpu.make_async_copy(rows0.at[pl.ds(0, ZT_B)],
                                      acc.at[pl.ds(r0, ZT_B)], g1).wait()

            plsc.subcore_barrier()

            pltpu.async_copy(h_hbm.at[srcb.at[0]], rows0, g0)

            def step(t, carry):
                j0 = 2 * t
                j1 = j0 + 1
                pltpu.make_async_copy(h_hbm.at[srcb.at[j0]], rows0, g0).wait()
                pltpu.async_copy(h_hbm.at[srcb.at[j1]], rows1, g1)
                pltpu.sync_copy(rows0, acc.at[dstb.at[j0]], add=True)
                pltpu.make_async_copy(h_hbm.at[srcb.at[j1]], rows1, g1).wait()
                pltpu.async_copy(h_hbm.at[srcb.at[j0 + 2]], rows0, g0)
                pltpu.sync_copy(rows1, acc.at[dstb.at[j1]], add=True)
                return carry

            lax.fori_loop(0, NBH - 1, step, 0)

            jL = NB - 2
            pltpu.make_async_copy(h_hbm.at[srcb.at[jL]], rows0, g0).wait()
            pltpu.async_copy(h_hbm.at[srcb.at[jL + 1]], rows1, g1)
            pltpu.sync_copy(rows0, acc.at[dstb.at[jL]], add=True)
            pltpu.make_async_copy(h_hbm.at[srcb.at[jL + 1]], rows1, g1).wait()
            pltpu.sync_copy(rows1, acc.at[dstb.at[jL + 1]], add=True)

            plsc.subcore_barrier()
            ob = chunk * 2 * N + c * N + s * RPT
            bufs = (rows0, rows1)
            sems = (g0, g1)
            for j in range(6):
                b = bufs[j % 2]
                sm = sems[j % 2]

                @pl.when(j < nfull)
                def _(j=j, b=b, sm=sm):
                    if j >= 2:
                        pltpu.make_async_copy(
                            b.at[pl.ds(0, ZCH)],
                            out_hbm.at[pl.ds(ob + (j - 2) * ZCH, ZCH)],
                            sm).wait()
                    pltpu.sync_copy(acc.at[pl.ds(r0 + j * ZCH, ZCH)],
                                    b.at[pl.ds(0, ZCH)])
                    pltpu.async_copy(b.at[pl.ds(0, ZCH)],
                                     out_hbm.at[pl.ds(ob + j * ZCH, ZCH)], sm)

            def wdrain(b, sm):
                pltpu.make_async_copy(b.at[pl.ds(0, ZCH)],
                                      out_hbm.at[pl.ds(ob, ZCH)], sm).wait()

            wdrain(rows0, g0)
            wdrain(rows1, g1)

            @pl.when(s < 15)
            def _():
                pltpu.sync_copy(acc.at[pl.ds(r0 + 6 * ZCH, ZT_A)],
                                rows0.at[pl.ds(0, ZT_A)])
                pltpu.sync_copy(rows0.at[pl.ds(0, ZT_A)],
                                out_hbm.at[pl.ds(ob + 6 * ZCH, ZT_A)])

            @pl.when(s == 15)
            def _():
                pltpu.sync_copy(acc.at[pl.ds(r0 + 5 * ZCH, ZT_B)],
                                rows0.at[pl.ds(0, ZT_B)])
                pltpu.sync_copy(rows0.at[pl.ds(0, ZT_B)],
                                out_hbm.at[pl.ds(ob + 5 * ZCH, ZT_B)])

    return pl.kernel(
        body,
        out_type=jax.ShapeDtypeStruct((2 * N, CH), jnp.float32),
        mesh=mesh,
        scratch_types=scratch,
    )


@functools.lru_cache(maxsize=None)
def _build_count():
    mesh = plsc.VectorSubcoreMesh(core_axis_name="c", subcore_axis_name="s")
    scratch = [
        pltpu.VMEM((NB, K), jnp.int32),
        pltpu.VMEM((K, CNTW), jnp.float32),
        pltpu.VMEM_SHARED((N, CNTW), jnp.float32),
        pltpu.VMEM((ZCH, CNTW), jnp.float32),
        pltpu.SemaphoreType.DMA,
        pltpu.VMEM((16,), jnp.int32),
        pltpu.VMEM((16, D), jnp.float32),
    ]

    def body(dst_hbm, zc_hbm, ones_hbm, table_hbm, ids_hbm,
             cnt_hbm, rows16_hbm,
             dstb, onesb, accc, cstage, sem, idxv, rowsv):
        c = lax.axis_index("c")
        s = lax.axis_index("s")
        tid = c * 16 + s
        r0 = s * RPT
        nfull = jnp.where(s < 15, 6, 5)
        pltpu.sync_copy(zc_hbm.at[pl.ds(0, ZCH)], cstage)
        pltpu.sync_copy(ones_hbm, onesb)
        pltpu.sync_copy(dst_hbm.at[tid], dstb)

        @pl.when(tid == 31)
        def _():
            pltpu.sync_copy(ids_hbm, idxv)
            pltpu.async_copy(table_hbm.at[idxv], rowsv, sem).wait()
            pltpu.sync_copy(rowsv, rows16_hbm)

        def zstep(j, carry):
            pltpu.sync_copy(cstage, accc.at[pl.ds(r0 + j * ZCH, ZCH)])
            return carry

        lax.fori_loop(0, nfull, zstep, 0)

        @pl.when(s < 15)
        def _():
            pltpu.sync_copy(cstage.at[pl.ds(0, ZT_A)],
                            accc.at[pl.ds(r0 + 6 * ZCH, ZT_A)])

        @pl.when(s == 15)
        def _():
            pltpu.sync_copy(cstage.at[pl.ds(0, ZT_B)],
                            accc.at[pl.ds(r0 + 5 * ZCH, ZT_B)])

        plsc.subcore_barrier()

        def rnd(r, carry):
            def issue(i, carry2):
                pltpu.async_copy(onesb, accc.at[dstb.at[r * 10 + i]], sem,
                                 add=True)
                return carry2

            lax.fori_loop(0, 10, issue, 0)

            def drain(i, carry2):
                pltpu.make_async_copy(onesb, accc.at[dstb.at[0]], sem).wait()
                return carry2

            lax.fori_loop(0, 10, drain, 0)
            return carry

        lax.fori_loop(0, NB // 10, rnd, 0)

        plsc.subcore_barrier()
        ob = c * N + s * RPT

        def wstep(j, carry):
            pltpu.sync_copy(accc.at[pl.ds(r0 + j * ZCH, ZCH)], cstage)
            pltpu.sync_copy(cstage, cnt_hbm.at[pl.ds(ob + j * ZCH, ZCH)])
            return carry

        lax.fori_loop(0, nfull, wstep, 0)

        @pl.when(s < 15)
        def _():
            pltpu.sync_copy(accc.at[pl.ds(r0 + 6 * ZCH, ZT_A)],
                            cstage.at[pl.ds(0, ZT_A)])
            pltpu.sync_copy(cstage.at[pl.ds(0, ZT_A)],
                            cnt_hbm.at[pl.ds(ob + 6 * ZCH, ZT_A)])

        @pl.when(s == 15)
        def _():
            pltpu.sync_copy(accc.at[pl.ds(r0 + 5 * ZCH, ZT_B)],
                            cstage.at[pl.ds(0, ZT_B)])
            pltpu.sync_copy(cstage.at[pl.ds(0, ZT_B)],
                            cnt_hbm.at[pl.ds(ob + 5 * ZCH, ZT_B)])

    return pl.kernel(
        body,
        out_type=[jax.ShapeDtypeStruct((2 * N, CNTW), jnp.float32),
                  jax.ShapeDtypeStruct((16, D), jnp.float32)],
        mesh=mesh,
        scratch_types=scratch,
    )


def _prologue(x, W, b):
    def body(x_ref, w_ref, b_ref, o_ref):
        o_ref[0] = (
            jnp.dot(x_ref[...], w_ref[...], preferred_element_type=jnp.float32)
            + b_ref[...]
        )

    return pl.pallas_call(
        body,
        grid=(NCH,),
        in_specs=[
            pl.BlockSpec((N, D_IN), lambda c: (0, 0)),
            pl.BlockSpec((D_IN, CH), lambda c: (0, c)),
            pl.BlockSpec((1, CH), lambda c: (0, c)),
        ],
        out_specs=pl.BlockSpec((1, N, CH), lambda c: (c, 0, 0)),
        out_shape=jax.ShapeDtypeStruct((NCH, N, CH), jnp.float32),
    )(x, W, b.reshape(1, D))


def _epilogue(h_c, part_c, cnt, gamma_c, beta_c, want_colsum):
    def body(h_ref, p_ref, c_ref, g_ref, bt_ref, o_ref):
        agg = p_ref[0] + p_ref[1]
        cntv = c_ref[0, :, 0:1] + c_ref[1, :, 0:1]
        hnew = h_ref[...] + agg / jnp.maximum(cntv, 1.0)
        mu = jnp.mean(hnew, axis=0, keepdims=True)
        var = jnp.mean((hnew - mu) ** 2, axis=0, keepdims=True)
        y = (hnew - mu) * lax.rsqrt(var + EPS) * g_ref[...] + bt_ref[...]
        y = jnp.maximum(y, 0.0)
        if want_colsum:
            o_ref[...] = jnp.sum(y, axis=0, keepdims=True)
        else:
            o_ref[...] = y

    out_shape = jax.ShapeDtypeStruct((1, CH) if want_colsum else (N, CH),
                                     jnp.float32)
    return pl.pallas_call(body, out_shape=out_shape)(
        h_c, part_c, cnt, gamma_c, beta_c)


def _head(qemb, rows16, W_fc, b_fc, g1, bt1, W2, b2, g2, bt2,
          W3, b3, g3, bt3, W4, b4):
    def bnrelu(t, g, bt):
        mu = jnp.mean(t, axis=0, keepdims=True)
        var = jnp.mean((t - mu) ** 2, axis=0, keepdims=True)
        return jnp.maximum((t - mu) * lax.rsqrt(var + EPS) * g + bt, 0.0)

    def body(q_ref, r_ref, wfc_ref, bfc_ref, g1_ref, bt1_ref,
             w2_ref, b2_ref, g2_ref, bt2_ref,
             w3_ref, b3_ref, g3_ref, bt3_ref,
             w4_ref, b4_ref, o_ref):
        q = jnp.broadcast_to(q_ref[...], (OUT_NUM, D))
        pg = jnp.broadcast_to(r_ref[10:11, :], (OUT_NUM, D))
        ne = r_ref[0:10, :]
        b = jnp.concatenate([q, pg, ne], axis=1)
        h1 = bnrelu(
            jnp.dot(b, wfc_ref[...], preferred_element_type=jnp.float32)
            + bfc_ref[...], g1_ref[...], bt1_ref[...])
        h2 = bnrelu(
            jnp.dot(h1, w2_ref[...], preferred_element_type=jnp.float32)
            + b2_ref[...], g2_ref[...], bt2_ref[...])
        h3 = bnrelu(
            jnp.dot(h2, w3_ref[...], preferred_element_type=jnp.float32)
            + b3_ref[...], g3_ref[...], bt3_ref[...])
        logits = jnp.dot(h3, w4_ref[...], preferred_element_type=jnp.float32) \
            + b4_ref[...]
        o_ref[...] = jax.nn.sigmoid(logits)

    return pl.pallas_call(
        body,
        out_shape=jax.ShapeDtypeStruct((OUT_NUM, 1), jnp.float32),
    )(qemb, rows16, W_fc, b_fc.reshape(1, -1), g1.reshape(1, -1),
      bt1.reshape(1, -1), W2, b2.reshape(1, -1), g2.reshape(1, -1),
      bt2.reshape(1, -1), W3, b3.reshape(1, -1), g3.reshape(1, -1),
      bt3.reshape(1, -1), W4, b4.reshape(1, -1))


def _conv_layer(h_chunks, src3, dst3, zeros_big):
    parts = []
    for c in range(NCH):
        p = _build_conv()(h_chunks[c], src3, dst3, zeros_big)
        if isinstance(p, (tuple, list)):
            p = p[0]
        parts.append(p.reshape(2, N, CH))
    return parts


def kernel(x, edge_index, neigh_ids, pg_ids, table, W_init, b_init,
           gamma1, beta1, gamma2, beta2,
           W_fc, b_fc, gamma_b1, beta_b1,
           W_fc2, b_fc2, gamma_b2, beta_b2,
           W_fc3, b_fc3, gamma_b3, beta_b3,
           W_fc4, b_fc4):
    src = edge_index[0]
    dst = edge_index[1]
    src3 = src.reshape(NTILES, NB, K)
    dst3 = dst.reshape(NTILES, NB, K)
    zeros_big = jnp.zeros((N, CH), jnp.float32)
    zeros_cnt = jnp.zeros((N, CNTW), jnp.float32)
    ones_arr = jnp.ones((K, CNTW), jnp.float32)

    h0 = _prologue(x, W_init, b_init)
    h0c = [h0[c] for c in range(NCH)]

    ids16 = jnp.concatenate(
        [neigh_ids, pg_ids, jnp.zeros((5,), jnp.int32)])
    cnt_raw, rows16 = _build_count()(dst3, zeros_cnt, ones_arr, table, ids16)
    cnt = cnt_raw.reshape(2, N, CNTW)

    parts1 = _conv_layer(h0c, src3, dst3, zeros_big)
    h1c = []
    for c in range(NCH):
        g = gamma1[c * CH:(c + 1) * CH].reshape(1, CH)
        bt = beta1[c * CH:(c + 1) * CH].reshape(1, CH)
        h1c.append(_epilogue(h0c[c], parts1[c], cnt, g, bt, False))

    parts2 = _conv_layer(h1c, src3, dst3, zeros_big)
    qs = []
    for c in range(NCH):
        g = gamma2[c * CH:(c + 1) * CH].reshape(1, CH)
        bt = beta2[c * CH:(c + 1) * CH].reshape(1, CH)
        qs.append(_epilogue(h1c[c], parts2[c], cnt, g, bt, True))
    qemb = jnp.concatenate(qs, axis=1) / float(N)

    pred = _head(qemb, rows16, W_fc, b_fc, gamma_b1, beta_b1,
                 W_fc2, b_fc2, gamma_b2, beta_b2,
                 W_fc3, b_fc3, gamma_b3, beta_b3,
                 W_fc4, b_fc4)
    return pred.reshape(1, OUT_NUM)
